# baseline (speedup 1.0000x reference)
import jax
import jax.numpy as jnp
from jax.experimental import pallas as pl
from jax.experimental.pallas import tpu as pltpu


def _scatter_body(idx_ref, upd_ref, in_ref, out_ref):
    del idx_ref, in_ref
    out_ref[...] = upd_ref[...]


def kernel(input, indices, update):
    D0, D1, D2 = input.shape
    K = indices.shape[0]

    grid_spec = pltpu.PrefetchScalarGridSpec(
        num_scalar_prefetch=1,
        grid=(K,),
        in_specs=[
            pl.BlockSpec((1, D1, D2), lambda i, idx_ref: (i, 0, 0)),
            pl.BlockSpec(memory_space=pl.ANY),
        ],
        out_specs=pl.BlockSpec((1, D1, D2), lambda i, idx_ref: (idx_ref[i], 0, 0)),
    )
    out = pl.pallas_call(
        _scatter_body,
        grid_spec=grid_spec,
        out_shape=jax.ShapeDtypeStruct((D0, D1, D2), input.dtype),
        input_output_aliases={2: 0},
    )(indices, update, input)
    return out


# speedup vs baseline: 16.3590x; 16.3590x over previous
import functools

import jax
import jax.numpy as jnp
from jax import lax
from jax.experimental import pallas as pl
from jax.experimental.pallas import tpu as pltpu
from jax.experimental.pallas import tpu_sc as plsc


def _make_sc_scatter(D0, K, R):
    info = plsc.get_sparse_core_info()
    NC, NS, L = info.num_cores, info.num_subcores, info.num_lanes
    NW = NC * NS
    CPW = K // NW
    CHUNK = L
    NCH = CPW // CHUNK
    PIECE = 2048
    NPIECE = K // PIECE
    VPP = PIECE // L
    POS_BITS = max(K - 1, 1).bit_length()
    POS_MASK = (1 << POS_BITS) - 1

    mesh = plsc.VectorSubcoreMesh(core_axis_name="c", subcore_axis_name="s")

    @functools.partial(
        pl.kernel,
        mesh=mesh,
        out_type=(),
        compiler_params=pltpu.CompilerParams(needs_layout_passes=False),
        scratch_types=[
            pltpu.VMEM((PIECE,), jnp.int32),
            pltpu.VMEM((NCH, CHUNK), jnp.int32),
            pltpu.VMEM((NCH, CHUNK), jnp.int32),
            pltpu.VMEM((D0,), jnp.int32),
            pltpu.VMEM((L,), jnp.int32),
            pltpu.VMEM((CHUNK, R), jnp.float32),
            pltpu.VMEM((CHUNK, R), jnp.float32),
            pltpu.SemaphoreType.DMA,
            pltpu.SemaphoreType.DMA,
            pltpu.SemaphoreType.DMA,
            pltpu.SemaphoreType.DMA,
        ],
    )
    def sc_scatter(idx_hbm, upd_hbm, out_hbm, piece_v, idx_own, jsrc, table,
                   shift_v, buf0, buf1, gsem0, gsem1, ssem0, ssem1):
        cid = lax.axis_index("c")
        sid = lax.axis_index("s")
        wid = sid * NC + cid
        base = wid * CPW
        my_piece = base // PIECE
        lanes = lax.iota(jnp.int32, L)
        last_lane = lanes == (L - 1)
        nxt_lane = jnp.minimum(lanes + 1, L - 1)

        def piece_body(p, _):
            pltpu.sync_copy(idx_hbm.at[pl.ds(p * PIECE, PIECE)], piece_v)

            def vreg_body(v, _):
                iv = piece_v[pl.ds(v * L, L)]
                pos = p * PIECE + v * L + lanes
                comb = (iv << POS_BITS) | pos
                ks, _ = plsc.sort_key_val(comb, comb)
                idx_s = ks >> POS_BITS
                pos_s = ks & POS_MASK
                shift_v[...] = idx_s
                nxt = plsc.load_gather(shift_v, [nxt_lane])
                keep = (idx_s != nxt) | last_lane
                plsc.store_scatter(table, [idx_s], pos_s, mask=keep)
                return 0

            lax.fori_loop(0, VPP, vreg_body, 0)

            @pl.when(p == my_piece)
            def _():
                off = base - my_piece * PIECE

                def cp_body(c, _):
                    idx_own[c] = piece_v[pl.ds(off + c * CHUNK, CHUNK)]
                    return 0

                lax.fori_loop(0, NCH, cp_body, 0)

            return 0

        lax.fori_loop(0, NPIECE, piece_body, 0)

        def src_body(c, _):
            iv = idx_own[c]
            jsrc[c] = plsc.load_gather(table, [iv])
            return 0

        lax.fori_loop(0, NCH, src_body, 0)

        bufs = (buf0, buf1)
        gsems = (gsem0, gsem1)
        ssems = (ssem0, ssem1)

        def pipe_body(cc, _):
            for b in range(2):
                c = cc * 2 + b

                @pl.when(cc > 0)
                def _():
                    pltpu.make_async_copy(
                        bufs[b], out_hbm.at[idx_own.at[0]], ssems[b]).wait()

                pltpu.async_copy(upd_hbm.at[jsrc.at[c]], bufs[b], gsems[b]).wait()
                pltpu.async_copy(bufs[b], out_hbm.at[idx_own.at[c]], ssems[b])
            return 0

        lax.fori_loop(0, NCH // 2, pipe_body, 0)
        for b in range(2):
            pltpu.make_async_copy(
                bufs[b], out_hbm.at[idx_own.at[0]], ssems[b]).wait()

    return sc_scatter


def kernel(input, indices, update):
    D0, D1, D2 = input.shape
    K = indices.shape[0]
    R = D1 * D2
    out_ref = jax.new_ref(input.reshape(D0, R))
    _make_sc_scatter(D0, K, R)(indices, update.reshape(K, R), out_ref)
    return jax.freeze(out_ref).reshape(D0, D1, D2)


# speedup vs baseline: 17.2626x; 1.0552x over previous
import functools

import jax
import jax.numpy as jnp
from jax import lax
from jax.experimental import pallas as pl
from jax.experimental.pallas import tpu as pltpu
from jax.experimental.pallas import tpu_sc as plsc

_SC_PARAMS = pltpu.CompilerParams(needs_layout_passes=False)


def _make_dedup(D0, K):
    info = plsc.get_sparse_core_info()
    NC, NS, L = info.num_cores, info.num_subcores, info.num_lanes
    NW = NC * NS
    CPW = K // NW
    PIECE = 2048
    NPIECE = K // PIECE
    VPP = PIECE // L
    POS_BITS = max(K - 1, 1).bit_length()
    POS_MASK = (1 << POS_BITS) - 1

    mesh = plsc.VectorSubcoreMesh(core_axis_name="c", subcore_axis_name="s")

    @functools.partial(
        pl.kernel,
        mesh=mesh,
        out_type=jax.ShapeDtypeStruct((K,), jnp.int32),
        compiler_params=_SC_PARAMS,
        scratch_types=[
            pltpu.VMEM((PIECE,), jnp.int32),
            pltpu.VMEM((D0,), jnp.int32),
            pltpu.VMEM((L,), jnp.int32),
            pltpu.VMEM((CPW,), jnp.int32),
            pltpu.VMEM((CPW,), jnp.int32),
        ],
    )
    def dedup(idx_hbm, jsrc_hbm, piece_v, table, shift_v, own_v, jbuf):
        cid = lax.axis_index("c")
        sid = lax.axis_index("s")
        wid = sid * NC + cid
        base = wid * CPW
        lanes = lax.iota(jnp.int32, L)
        last_lane = lanes == (L - 1)
        nxt_lane = jnp.minimum(lanes + 1, L - 1)

        def piece_body(p, _):
            pltpu.sync_copy(idx_hbm.at[pl.ds(p * PIECE, PIECE)], piece_v)

            def vreg_body(v, _):
                iv = piece_v[pl.ds(v * L, L)]
                pos = p * PIECE + v * L + lanes
                comb = (iv << POS_BITS) | pos
                ks, _ = plsc.sort_key_val(comb, comb)
                idx_s = ks >> POS_BITS
                pos_s = ks & POS_MASK
                shift_v[...] = idx_s
                nxt = plsc.load_gather(shift_v, [nxt_lane])
                keep = (idx_s != nxt) | last_lane
                plsc.store_scatter(table, [idx_s], pos_s, mask=keep)
                return 0

            lax.fori_loop(0, VPP, vreg_body, 0)
            return 0

        lax.fori_loop(0, NPIECE, piece_body, 0)

        pltpu.sync_copy(idx_hbm.at[pl.ds(base, CPW)], own_v)

        def src_body(c, _):
            iv = own_v[pl.ds(c * L, L)]
            jbuf[pl.ds(c * L, L)] = plsc.load_gather(table, [iv])
            return 0

        lax.fori_loop(0, CPW // L, src_body, 0)
        pltpu.sync_copy(jbuf, jsrc_hbm.at[pl.ds(base, CPW)])

    return dedup


def _make_scatter(D0, K, R):
    info = plsc.get_sparse_core_info()
    NC, NS, L = info.num_cores, info.num_subcores, info.num_lanes
    NW = NC * NS
    CPW = K // NW
    CHUNK = L
    NCH = CPW // CHUNK

    mesh = plsc.VectorSubcoreMesh(core_axis_name="c", subcore_axis_name="s")

    @functools.partial(
        pl.kernel,
        mesh=mesh,
        out_type=(),
        compiler_params=_SC_PARAMS,
        scratch_types=[
            pltpu.VMEM((CPW,), jnp.int32),
            pltpu.VMEM((NCH, CHUNK), jnp.int32),
            pltpu.VMEM((NCH, CHUNK), jnp.int32),
            pltpu.VMEM((CHUNK, R), jnp.float32),
            pltpu.VMEM((CHUNK, R), jnp.float32),
            pltpu.SemaphoreType.DMA,
            pltpu.SemaphoreType.DMA,
            pltpu.SemaphoreType.DMA,
            pltpu.SemaphoreType.DMA,
        ],
    )
    def scatter(idx_hbm, jsrc_hbm, upd_hbm, out_hbm, stage_v, idx_own, jsrc,
                buf0, buf1, gsem0, gsem1, ssem0, ssem1):
        cid = lax.axis_index("c")
        sid = lax.axis_index("s")
        wid = sid * NC + cid
        base = wid * CPW

        pltpu.sync_copy(idx_hbm.at[pl.ds(base, CPW)], stage_v)

        def repack_idx(c, _):
            idx_own[c] = stage_v[pl.ds(c * CHUNK, CHUNK)]
            return 0

        lax.fori_loop(0, NCH, repack_idx, 0)
        pltpu.sync_copy(jsrc_hbm.at[pl.ds(base, CPW)], stage_v)

        def repack_j(c, _):
            jsrc[c] = stage_v[pl.ds(c * CHUNK, CHUNK)]
            return 0

        lax.fori_loop(0, NCH, repack_j, 0)

        bufs = (buf0, buf1)
        gsems = (gsem0, gsem1)
        ssems = (ssem0, ssem1)

        def pipe_body(cc, _):
            for b in range(2):
                c = cc * 2 + b

                @pl.when(cc > 0)
                def _():
                    pltpu.make_async_copy(
                        bufs[b], out_hbm.at[idx_own.at[0]], ssems[b]).wait()

                pltpu.async_copy(upd_hbm.at[jsrc.at[c]], bufs[b], gsems[b]).wait()
                pltpu.async_copy(bufs[b], out_hbm.at[idx_own.at[c]], ssems[b])
            return 0

        lax.fori_loop(0, NCH // 2, pipe_body, 0)
        for b in range(2):
            pltpu.make_async_copy(
                bufs[b], out_hbm.at[idx_own.at[0]], ssems[b]).wait()

    return scatter


def kernel(input, indices, update):
    D0, D1, D2 = input.shape
    K = indices.shape[0]
    R = D1 * D2
    jsrc = _make_dedup(D0, K)(indices)
    out_ref = jax.new_ref(input.reshape(D0, R))
    _make_scatter(D0, K, R)(indices, jsrc, update.reshape(K, R), out_ref)
    return jax.freeze(out_ref).reshape(D0, D1, D2)
